# Initial kernel scaffold; baseline (speedup 1.0000x reference)
import jax
import jax.numpy as jnp
from jax.experimental import pallas as pl


def kernel(tokens, positions, tok_table, pos_table):
    raise NotImplementedError("write your pallas kernel here")


# speedup vs baseline: 1.2880x; 1.2880x over previous
import functools

import jax
import jax.numpy as jnp
from jax import lax
from jax.experimental import pallas as pl
from jax.experimental.pallas import tpu as pltpu
from jax.experimental.pallas import tpu_sc as plsc

VOCAB = 100000
EMBED = 768
SEQ = 2048
BATCH = 4

NUM_CORES = 2
NUM_SUBCORES = 16
NW = NUM_CORES * NUM_SUBCORES
ROWS = BATCH * SEQ
R_PER_W = ROWS // NW
C = 64
NCHUNK = R_PER_W // C
LANES = 16
NVEC = EMBED // LANES

_mesh = plsc.VectorSubcoreMesh(core_axis_name="c", subcore_axis_name="s")


@functools.partial(
    pl.kernel,
    mesh=_mesh,
    out_type=jax.ShapeDtypeStruct((ROWS, EMBED), jnp.float32),
    scratch_types=[
        pltpu.VMEM((C,), jnp.int32),
        pltpu.VMEM((C,), jnp.int32),
        pltpu.VMEM((C, EMBED), jnp.float32),
        pltpu.VMEM((C, EMBED), jnp.float32),
        pltpu.SemaphoreType.DMA,
        pltpu.SemaphoreType.DMA,
    ],
)
def _embed_sc(tok_hbm, pos_hbm, tok_table, pos_table, out_hbm,
              tok_idx, pos_idx, buf_a, buf_b, sem_a, sem_b):
    wid = lax.axis_index("s") * NUM_CORES + lax.axis_index("c")
    base = wid * R_PER_W
    for c in range(NCHUNK):
        off = pl.multiple_of(base + c * C, C)
        pltpu.sync_copy(tok_hbm.at[pl.ds(off, C)], tok_idx)
        pltpu.sync_copy(pos_hbm.at[pl.ds(off, C)], pos_idx)
        cp_a = pltpu.async_copy(tok_table.at[tok_idx], buf_a, sem_a)
        cp_b = pltpu.async_copy(pos_table.at[pos_idx], buf_b, sem_b)
        cp_a.wait()
        cp_b.wait()

        def row(i, carry):
            for j in range(NVEC):
                x = buf_b[i, pl.ds(j * LANES, LANES)]
                plsc.addupdate(buf_a.at[i, pl.ds(j * LANES, LANES)], x)
            return carry

        lax.fori_loop(0, C, row, 0)
        pltpu.sync_copy(buf_a, out_hbm.at[pl.ds(off, C)])


def kernel(tokens, positions, tok_table, pos_table):
    tok_flat = tokens.reshape(ROWS).astype(jnp.int32)
    pos_flat = positions.reshape(ROWS).astype(jnp.int32)
    out = _embed_sc(tok_flat, pos_flat, tok_table, pos_table)
    return out.reshape(BATCH, SEQ, EMBED)


# speedup vs baseline: 1.5757x; 1.2233x over previous
import functools

import jax
import jax.numpy as jnp
from jax import lax
from jax.experimental import pallas as pl
from jax.experimental.pallas import tpu as pltpu
from jax.experimental.pallas import tpu_sc as plsc

VOCAB = 100000
EMBED = 768
SEQ = 2048
BATCH = 4

NUM_CORES = 2
NUM_SUBCORES = 16
NW = NUM_CORES * NUM_SUBCORES
ROWS = BATCH * SEQ
R_PER_W = ROWS // NW
C = 32
NCHUNK = R_PER_W // C
LANES = 16
NVEC = EMBED // LANES

_mesh = plsc.VectorSubcoreMesh(core_axis_name="c", subcore_axis_name="s")


@functools.partial(
    pl.kernel,
    mesh=_mesh,
    out_type=jax.ShapeDtypeStruct((ROWS, EMBED), jnp.float32),
    scratch_types=[
        pltpu.VMEM((R_PER_W,), jnp.int32),
        pltpu.VMEM((R_PER_W,), jnp.int32),
        pltpu.VMEM((C, EMBED), jnp.float32),
        pltpu.VMEM((C, EMBED), jnp.float32),
        pltpu.VMEM((C, EMBED), jnp.float32),
        pltpu.VMEM((C, EMBED), jnp.float32),
        pltpu.SemaphoreType.DMA,
        pltpu.SemaphoreType.DMA,
        pltpu.SemaphoreType.DMA,
        pltpu.SemaphoreType.DMA,
        pltpu.SemaphoreType.DMA,
        pltpu.SemaphoreType.DMA,
    ],
)
def _embed_sc(tok_hbm, pos_hbm, tok_table, pos_table, out_hbm,
              tok_idx, pos_idx, buf_a0, buf_a1, buf_b0, buf_b1,
              sem_a0, sem_a1, sem_b0, sem_b1, sem_o0, sem_o1):
    wid = lax.axis_index("s") * NUM_CORES + lax.axis_index("c")
    base = wid * R_PER_W
    pltpu.sync_copy(tok_hbm.at[pl.ds(base, R_PER_W)], tok_idx)
    pltpu.sync_copy(pos_hbm.at[pl.ds(base, R_PER_W)], pos_idx)

    bufs_a = [buf_a0, buf_a1]
    bufs_b = [buf_b0, buf_b1]
    sems_a = [sem_a0, sem_a1]
    sems_b = [sem_b0, sem_b1]
    sems_o = [sem_o0, sem_o1]

    def fire(c):
        s = c & 1
        ga = pltpu.async_copy(
            tok_table.at[tok_idx.at[pl.ds(c * C, C)]], bufs_a[s], sems_a[s])
        gb = pltpu.async_copy(
            pos_table.at[pos_idx.at[pl.ds(c * C, C)]], bufs_b[s], sems_b[s])
        return ga, gb

    gcp = [None, None]
    ocp = [None, None]
    gcp[0] = fire(0)
    for c in range(NCHUNK):
        s = c & 1
        s2 = 1 - s
        if c + 1 < NCHUNK:
            if ocp[s2] is not None:
                ocp[s2].wait()
            gcp[s2] = fire(c + 1)
        ga, gb = gcp[s]
        ga.wait()
        gb.wait()

        def row(i, carry, s=s):
            for j in range(NVEC):
                x = bufs_b[s][i, pl.ds(j * LANES, LANES)]
                plsc.addupdate(bufs_a[s].at[i, pl.ds(j * LANES, LANES)], x)
            return carry

        lax.fori_loop(0, C, row, 0)
        ocp[s] = pltpu.async_copy(
            bufs_a[s], out_hbm.at[pl.ds(base + c * C, C)], sems_o[s])
    ocp[0].wait()
    ocp[1].wait()


def kernel(tokens, positions, tok_table, pos_table):
    tok_flat = tokens.reshape(ROWS).astype(jnp.int32)
    pos_flat = positions.reshape(ROWS).astype(jnp.int32)
    out = _embed_sc(tok_flat, pos_flat, tok_table, pos_table)
    return out.reshape(BATCH, SEQ, EMBED)


# speedup vs baseline: 1.5820x; 1.0040x over previous
import functools

import jax
import jax.numpy as jnp
from jax import lax
from jax.experimental import pallas as pl
from jax.experimental.pallas import tpu as pltpu
from jax.experimental.pallas import tpu_sc as plsc

VOCAB = 100000
EMBED = 768
SEQ = 2048
BATCH = 4

NUM_CORES = 2
NUM_SUBCORES = 16
NW = NUM_CORES * NUM_SUBCORES
ROWS = BATCH * SEQ
R_PER_W = ROWS // NW
C = 32
NCHUNK = R_PER_W // C
LANES = 16
NVEC = EMBED // LANES

_mesh = plsc.VectorSubcoreMesh(core_axis_name="c", subcore_axis_name="s")


W_PER_B = SEQ // R_PER_W


@functools.partial(
    pl.kernel,
    mesh=_mesh,
    out_type=jax.ShapeDtypeStruct((BATCH, SEQ, EMBED), jnp.float32),
    scratch_types=[
        pltpu.VMEM((R_PER_W,), jnp.int32),
        pltpu.VMEM((R_PER_W,), jnp.int32),
        pltpu.VMEM((C, EMBED), jnp.float32),
        pltpu.VMEM((C, EMBED), jnp.float32),
        pltpu.VMEM((C, EMBED), jnp.float32),
        pltpu.VMEM((C, EMBED), jnp.float32),
        pltpu.SemaphoreType.DMA,
        pltpu.SemaphoreType.DMA,
        pltpu.SemaphoreType.DMA,
        pltpu.SemaphoreType.DMA,
        pltpu.SemaphoreType.DMA,
        pltpu.SemaphoreType.DMA,
    ],
)
def _embed_sc(tok_hbm, pos_hbm, tok_table, pos_table, out_hbm,
              tok_idx, pos_idx, buf_a0, buf_a1, buf_b0, buf_b1,
              sem_a0, sem_a1, sem_b0, sem_b1, sem_o0, sem_o1):
    wid = lax.axis_index("s") * NUM_CORES + lax.axis_index("c")
    b = wid // W_PER_B
    s0 = (wid % W_PER_B) * R_PER_W
    pltpu.sync_copy(tok_hbm.at[b, pl.ds(s0, R_PER_W)], tok_idx)
    pltpu.sync_copy(pos_hbm.at[b, pl.ds(s0, R_PER_W)], pos_idx)

    bufs_a = [buf_a0, buf_a1]
    bufs_b = [buf_b0, buf_b1]
    sems_a = [sem_a0, sem_a1]
    sems_b = [sem_b0, sem_b1]
    sems_o = [sem_o0, sem_o1]

    def fire(c):
        s = c & 1
        ga = pltpu.async_copy(
            tok_table.at[tok_idx.at[pl.ds(c * C, C)]], bufs_a[s], sems_a[s])
        gb = pltpu.async_copy(
            pos_table.at[pos_idx.at[pl.ds(c * C, C)]], bufs_b[s], sems_b[s])
        return ga, gb

    gcp = [None, None]
    ocp = [None, None]
    gcp[0] = fire(0)
    for c in range(NCHUNK):
        s = c & 1
        s2 = 1 - s
        if c + 1 < NCHUNK:
            if ocp[s2] is not None:
                ocp[s2].wait()
            gcp[s2] = fire(c + 1)
        ga, gb = gcp[s]
        ga.wait()
        gb.wait()

        def row(i, carry, s=s):
            for j in range(NVEC):
                x = bufs_b[s][i, pl.ds(j * LANES, LANES)]
                plsc.addupdate(bufs_a[s].at[i, pl.ds(j * LANES, LANES)], x)
            return carry

        lax.fori_loop(0, C, row, 0)
        ocp[s] = pltpu.async_copy(
            bufs_a[s], out_hbm.at[b, pl.ds(s0 + c * C, C)], sems_o[s])
    ocp[0].wait()
    ocp[1].wait()


def kernel(tokens, positions, tok_table, pos_table):
    return _embed_sc(tokens, positions, tok_table, pos_table)
